# baseline (speedup 1.0000x reference)
import functools
import math

import jax
import jax.numpy as jnp
from jax import lax
from jax.experimental import pallas as pl
from jax.experimental.pallas import tpu as pltpu
from jax.experimental.pallas import tpu_sc as plsc

B = 4
SEQ = 2048
D_F = 1024
SCALE = math.sqrt(D_F)

NC = 2
NS = 16
NW = NC * NS
S_PER_W = SEQ // NW
LANES = 16
D_VECS = D_F // LANES

CH = 4
NBUF = 4
CPB = S_PER_W // CH
N_CHUNK = B * CPB
N_GROUP = N_CHUNK // NBUF


def _sc_body(x_hbm, idx_hbm, pe_hbm, seg_hbm, out_hbm,
             idx_v, seg_v, pe_v, xb, ob,
             sem_pe, sem_seg, sem_in, sem_out):
    wid = lax.axis_index("s") * NC + lax.axis_index("c")
    s0 = wid * S_PER_W

    pltpu.sync_copy(idx_hbm, idx_v)
    seg_cp = pltpu.async_copy(seg_hbm.at[idx_v], seg_v, sem_seg)
    pe_cp = pltpu.async_copy(pe_hbm.at[0, pl.ds(s0, S_PER_W)], pe_v, sem_pe)
    seg_cp.wait()
    pe_cp.wait()

    def fold_row(r, _):
        for j in range(D_VECS):
            sl = pl.ds(j * LANES, LANES)
            pe_v[r, sl] = pe_v[r, sl] + seg_v[0, sl]
        return 0

    lax.fori_loop(0, S_PER_W, fold_row, 0, unroll=False)

    def chunk_src(i):
        b = i // CPB
        soff = (i % CPB) * CH
        return x_hbm.at[b, pl.ds(s0 + soff, CH)], b, soff

    for k in range(NBUF):
        src, _, _ = chunk_src(k)
        pltpu.async_copy(src, xb.at[k], sem_in.at[k])

    def group(g, _):
        for k in range(NBUF):
            i = g * NBUF + k
            pltpu.make_async_copy(x_hbm.at[0, pl.ds(0, CH)], xb.at[k],
                                  sem_in.at[k]).wait()

            @pl.when(g > 0)
            def _wait_out():
                pltpu.make_async_copy(ob.at[k], out_hbm.at[0, pl.ds(0, CH)],
                                      sem_out.at[k]).wait()

            b = i // CPB
            soff = (i % CPB) * CH

            xv = xb.at[k]
            ov = ob.at[k]

            def row(r, _):
                for j in range(D_VECS):
                    sl = pl.ds(j * LANES, LANES)
                    ov[r, sl] = xv[r, sl] * SCALE + pe_v[soff + r, sl]
                return 0

            lax.fori_loop(0, CH, row, 0, unroll=False)
            pltpu.async_copy(ob.at[k], out_hbm.at[b, pl.ds(s0 + soff, CH)],
                             sem_out.at[k])

            @pl.when(g < N_GROUP - 1)
            def _fire_next_in():
                ni = (g + 1) * NBUF + k
                nb = ni // CPB
                nsoff = (ni % CPB) * CH
                pltpu.async_copy(x_hbm.at[nb, pl.ds(s0 + nsoff, CH)],
                                 xb.at[k], sem_in.at[k])
        return 0

    lax.fori_loop(0, N_GROUP, group, 0, unroll=False)

    for k in range(NBUF):
        pltpu.make_async_copy(ob.at[k], out_hbm.at[0, pl.ds(0, CH)],
                              sem_out.at[k]).wait()


@jax.jit
def _pos_encode_sc(x, seg_idx, pe, seg_table):
    mesh = plsc.VectorSubcoreMesh(core_axis_name="c", subcore_axis_name="s")
    kfn = functools.partial(
        pl.kernel,
        mesh=mesh,
        out_type=jax.ShapeDtypeStruct((B, SEQ, D_F), jnp.float32),
        scratch_types=[
            pltpu.VMEM((8,), jnp.int32),
            pltpu.VMEM((8, D_F), jnp.float32),
            pltpu.VMEM((S_PER_W, D_F), jnp.float32),
            pltpu.VMEM((NBUF, CH, D_F), jnp.float32),
            pltpu.VMEM((NBUF, CH, D_F), jnp.float32),
            pltpu.SemaphoreType.DMA,
            pltpu.SemaphoreType.DMA,
            pltpu.SemaphoreType.DMA((NBUF,)),
            pltpu.SemaphoreType.DMA((NBUF,)),
        ],
    )(_sc_body)
    return kfn(x, seg_idx, pe, seg_table)


def kernel(x, view_idx, pe, seg_table):
    seq_len = x.shape[1]
    seg_idx = jnp.full((8,), view_idx * seq_len, dtype=jnp.int32)
    return _pos_encode_sc(x, seg_idx, pe, seg_table)


# speedup vs baseline: 4.0103x; 4.0103x over previous
import functools
import math

import jax
import jax.numpy as jnp
from jax import lax
from jax.experimental import pallas as pl
from jax.experimental.pallas import tpu as pltpu
from jax.experimental.pallas import tpu_sc as plsc

B = 4
SEQ = 2048
D_F = 1024
SCALE = math.sqrt(D_F)

NC = 2
TS = 512


def _sc_gather_body(idx_hbm, seg_hbm, out_hbm, idx_v, seg_v, sem):
    wid = lax.axis_index("s") * NC + lax.axis_index("c")

    @pl.when(wid == 0)
    def _():
        pltpu.sync_copy(idx_hbm, idx_v)
        cp = pltpu.async_copy(seg_hbm.at[idx_v], seg_v, sem)
        cp.wait()
        pltpu.sync_copy(seg_v, out_hbm)


def _tc_body(x_ref, pe_ref, seg_ref, o_ref):
    o_ref[...] = x_ref[...] * SCALE + (pe_ref[...] + seg_ref[0, :][None, None, :])


@jax.jit
def _pos_encode(x, seg_idx, pe, seg_table):
    mesh = plsc.VectorSubcoreMesh(core_axis_name="c", subcore_axis_name="s")
    seg_row = pl.kernel(
        _sc_gather_body,
        mesh=mesh,
        out_type=jax.ShapeDtypeStruct((8, D_F), jnp.float32),
        scratch_types=[
            pltpu.VMEM((8,), jnp.int32),
            pltpu.VMEM((8, D_F), jnp.float32),
            pltpu.SemaphoreType.DMA,
        ],
    )(seg_idx, seg_table)

    return pl.pallas_call(
        _tc_body,
        grid=(SEQ // TS, B),
        in_specs=[
            pl.BlockSpec((1, TS, D_F), lambda i, b: (b, i, 0)),
            pl.BlockSpec((1, TS, D_F), lambda i, b: (0, i, 0)),
            pl.BlockSpec((8, D_F), lambda i, b: (0, 0)),
        ],
        out_specs=pl.BlockSpec((1, TS, D_F), lambda i, b: (b, i, 0)),
        out_shape=jax.ShapeDtypeStruct((B, SEQ, D_F), jnp.float32),
    )(x, pe, seg_row)


def kernel(x, view_idx, pe, seg_table):
    seq_len = x.shape[1]
    seg_idx = jnp.full((8,), view_idx * seq_len, dtype=jnp.int32)
    return _pos_encode(x, seg_idx, pe, seg_table)


# speedup vs baseline: 4.2148x; 1.0510x over previous
import functools
import math

import jax
import jax.numpy as jnp
from jax import lax
from jax.experimental import pallas as pl
from jax.experimental.pallas import tpu as pltpu
from jax.experimental.pallas import tpu_sc as plsc

B = 4
SEQ = 2048
D_F = 1024
SCALE = math.sqrt(D_F)

NC = 2
TS = 1024


def _sc_gather_body(idx_hbm, seg_hbm, out_hbm, idx_v, seg_v, sem):
    wid = lax.axis_index("s") * NC + lax.axis_index("c")

    @pl.when(wid == 0)
    def _():
        pltpu.sync_copy(idx_hbm, idx_v)
        cp = pltpu.async_copy(seg_hbm.at[idx_v], seg_v, sem)
        cp.wait()
        pltpu.sync_copy(seg_v, out_hbm)


def _tc_body(x_ref, pe_ref, seg_ref, o_ref):
    o_ref[...] = x_ref[...] * SCALE + (pe_ref[...] + seg_ref[0, :][None, None, :])


@jax.jit
def _pos_encode(x, seg_idx, pe, seg_table):
    mesh = plsc.VectorSubcoreMesh(core_axis_name="c", subcore_axis_name="s")
    seg_row = pl.kernel(
        _sc_gather_body,
        mesh=mesh,
        out_type=jax.ShapeDtypeStruct((8, D_F), jnp.float32),
        scratch_types=[
            pltpu.VMEM((8,), jnp.int32),
            pltpu.VMEM((8, D_F), jnp.float32),
            pltpu.SemaphoreType.DMA,
        ],
    )(seg_idx, seg_table)

    return pl.pallas_call(
        _tc_body,
        grid=(SEQ // TS, B),
        in_specs=[
            pl.BlockSpec((1, TS, D_F), lambda i, b: (b, i, 0)),
            pl.BlockSpec((1, TS, D_F), lambda i, b: (0, i, 0)),
            pl.BlockSpec((8, D_F), lambda i, b: (0, 0)),
        ],
        out_specs=pl.BlockSpec((1, TS, D_F), lambda i, b: (b, i, 0)),
        out_shape=jax.ShapeDtypeStruct((B, SEQ, D_F), jnp.float32),
    )(x, pe, seg_row)


def kernel(x, view_idx, pe, seg_table):
    seq_len = x.shape[1]
    seg_idx = jnp.full((8,), view_idx * seq_len, dtype=jnp.int32)
    return _pos_encode(x, seg_idx, pe, seg_table)


# speedup vs baseline: 4.3910x; 1.0418x over previous
import functools
import math

import jax
import jax.numpy as jnp
from jax import lax
from jax.experimental import pallas as pl
from jax.experimental.pallas import tpu as pltpu
from jax.experimental.pallas import tpu_sc as plsc

B = 4
SEQ = 2048
D_F = 1024
SCALE = math.sqrt(D_F)

NC = 2
TS = 2048


def _sc_gather_body(idx_hbm, seg_hbm, out_hbm, idx_v, seg_v, sem):
    wid = lax.axis_index("s") * NC + lax.axis_index("c")

    @pl.when(wid == 0)
    def _():
        pltpu.sync_copy(idx_hbm, idx_v)
        cp = pltpu.async_copy(seg_hbm.at[idx_v], seg_v, sem)
        cp.wait()
        pltpu.sync_copy(seg_v, out_hbm)


def _tc_body(x_ref, pe_ref, seg_ref, o_ref):
    o_ref[...] = x_ref[...] * SCALE + (pe_ref[...] + seg_ref[0, :][None, None, :])


@jax.jit
def _pos_encode(x, seg_idx, pe, seg_table):
    mesh = plsc.VectorSubcoreMesh(core_axis_name="c", subcore_axis_name="s")
    seg_row = pl.kernel(
        _sc_gather_body,
        mesh=mesh,
        out_type=jax.ShapeDtypeStruct((8, D_F), jnp.float32),
        scratch_types=[
            pltpu.VMEM((8,), jnp.int32),
            pltpu.VMEM((8, D_F), jnp.float32),
            pltpu.SemaphoreType.DMA,
        ],
    )(seg_idx, seg_table)

    return pl.pallas_call(
        _tc_body,
        grid=(SEQ // TS, B),
        in_specs=[
            pl.BlockSpec((1, TS, D_F), lambda i, b: (b, i, 0)),
            pl.BlockSpec((1, TS, D_F), lambda i, b: (0, i, 0)),
            pl.BlockSpec((8, D_F), lambda i, b: (0, 0)),
        ],
        out_specs=pl.BlockSpec((1, TS, D_F), lambda i, b: (b, i, 0)),
        out_shape=jax.ShapeDtypeStruct((B, SEQ, D_F), jnp.float32),
    )(x, pe, seg_row)


def kernel(x, view_idx, pe, seg_table):
    seq_len = x.shape[1]
    seg_idx = jnp.full((8,), view_idx * seq_len, dtype=jnp.int32)
    return _pos_encode(x, seg_idx, pe, seg_table)
